# baseline (speedup 1.0000x reference)
import functools

import jax
import jax.numpy as jnp
from jax import lax
from jax.experimental import pallas as pl
from jax.experimental.pallas import tpu as pltpu
from jax.experimental.pallas import tpu_sc as plsc

N = 50000
E = 800000
NC = 2
NS = 16
L = 16
W = 16
LCHUNK = 1600
MCHUNK = 1280
NCH = E // MCHUNK
BN = 2048

_SC_PARAMS = pltpu.CompilerParams(
    needs_layout_passes=False, use_tc_tiling_on_sc=False)


def _cdiv(a, b):
    return (a + b - 1) // b


def _make_logit_kernel():
    mesh = plsc.VectorSubcoreMesh(core_axis_name="c", subcore_axis_name="s")
    nw = NC * NS
    lnch = E // LCHUNK
    iters = _cdiv(lnch, nw)

    @functools.partial(
        pl.kernel,
        out_type=jax.ShapeDtypeStruct((E,), jnp.float32),
        mesh=mesh,
        compiler_params=_SC_PARAMS,
        scratch_types=[
            pltpu.VMEM((N,), jnp.float32),
            pltpu.VMEM((N,), jnp.float32),
            pltpu.VMEM((LCHUNK,), jnp.int32),
            pltpu.VMEM((LCHUNK,), jnp.int32),
            pltpu.VMEM((LCHUNK,), jnp.float32),
        ],
    )
    def k(asrc_hbm, adst_hbm, src_hbm, dst_hbm, s_hbm, asrc_v, adst_v,
          srcv, dstv, sv):
        c = lax.axis_index("c")
        sid = lax.axis_index("s")
        wid = sid * NC + c
        pltpu.sync_copy(asrc_hbm, asrc_v)
        pltpu.sync_copy(adst_hbm, adst_v)

        def chunk_body(kk, carry):
            cid = wid + nw * kk

            @pl.when(cid < lnch)
            def _():
                off = cid * LCHUNK
                pltpu.sync_copy(src_hbm.at[pl.ds(off, LCHUNK)], srcv)
                pltpu.sync_copy(dst_hbm.at[pl.ds(off, LCHUNK)], dstv)

                def g_body(g, carry2):
                    si = srcv[pl.ds(g * L, L)]
                    di = dstv[pl.ds(g * L, L)]
                    a = plsc.load_gather(asrc_v, [si])
                    b = plsc.load_gather(adst_v, [di])
                    e = a + b
                    e = jnp.where(e >= 0.0, e, 0.2 * e)
                    e = jnp.minimum(e, 60.0)
                    sv[pl.ds(g * L, L)] = jnp.exp(e)
                    return carry2

                lax.fori_loop(0, LCHUNK // L, g_body, 0)
                pltpu.sync_copy(sv, s_hbm.at[pl.ds(off, LCHUNK)])

            return carry

        lax.fori_loop(0, iters, chunk_body, 0)

    return k


def _make_msg_kernel(n_tables, schedules, n_out):
    mesh = plsc.VectorSubcoreMesh(core_axis_name="c", subcore_axis_name="s")
    zrows = 400
    nz = N // zrows
    z_iters = _cdiv(nz, NS)

    scratch = [pltpu.VMEM_SHARED((N, W), jnp.float32),
               pltpu.VMEM((MCHUNK, W), jnp.float32)]
    for _ in range(2):
        scratch += [
            pltpu.VMEM((MCHUNK,), jnp.int32),
            pltpu.VMEM((MCHUNK,), jnp.int32),
            pltpu.VMEM((MCHUNK,), jnp.float32),
            pltpu.VMEM((MCHUNK, W), jnp.float32),
            pltpu.SemaphoreType.DMA,
            pltpu.SemaphoreType.DMA,
        ]

    @functools.partial(
        pl.kernel,
        out_type=jax.ShapeDtypeStruct((n_out, N, W), jnp.float32),
        mesh=mesh,
        compiler_params=_SC_PARAMS,
        scratch_types=scratch,
    )
    def k(*refs):
        tabs = refs[:n_tables]
        src_hbm, dst_hbm, s_hbm, out_hbm, acc, scaled = (
            refs[n_tables:n_tables + 6])
        slots = [refs[n_tables + 6 + 6 * b:n_tables + 6 + 6 * (b + 1)]
                 for b in range(2)]
        c = lax.axis_index("c")
        sid = lax.axis_index("s")

        def issue_idx(cid, b):
            srcv, dstv, sv, _, sem_i, _ = slots[b]
            off = cid * MCHUNK
            pltpu.async_copy(src_hbm.at[pl.ds(off, MCHUNK)], srcv, sem_i)
            pltpu.async_copy(dst_hbm.at[pl.ds(off, MCHUNK)], dstv, sem_i)
            pltpu.async_copy(s_hbm.at[pl.ds(off, MCHUNK)], sv, sem_i)

        def wait_idx(cid, b):
            srcv, dstv, sv, _, sem_i, _ = slots[b]
            off = cid * MCHUNK
            pltpu.make_async_copy(
                src_hbm.at[pl.ds(off, MCHUNK)], srcv, sem_i).wait()
            pltpu.make_async_copy(
                dst_hbm.at[pl.ds(off, MCHUNK)], dstv, sem_i).wait()
            pltpu.make_async_copy(
                s_hbm.at[pl.ds(off, MCHUNK)], sv, sem_i).wait()

        def do_pass(tab_hbm, lo, hi, out_idx):
            iters = _cdiv(hi - lo, NS)

            rows0 = slots[0][3]

            def zb(i, carry):
                rows0[i, :] = jnp.zeros((L,), jnp.float32)
                return carry

            lax.fori_loop(0, zrows, zb, 0)
            for t in range(z_iters):
                z = sid + NS * t

                @pl.when(z < nz)
                def _():
                    pltpu.sync_copy(rows0.at[pl.ds(0, zrows)],
                                    acc.at[pl.ds(z * zrows, zrows)])
            plsc.subcore_barrier()

            cid0 = lo + sid

            @pl.when(cid0 < hi)
            def _():
                issue_idx(cid0, 0)

            def chunk_pair(kk2, carry):
                for b in (0, 1):
                    kk2b = 2 * kk2 + b
                    cid = lo + sid + NS * kk2b
                    prev = cid - NS
                    nxt = cid + NS
                    srcv, dstv, sv, rows, sem_i, sem_g = slots[b]
                    srcp, dstp, svp, rowsp, _, sem_gp = slots[1 - b]

                    @pl.when(cid < hi)
                    def _():
                        wait_idx(cid, b)
                        pltpu.async_copy(tab_hbm.at[srcv], rows, sem_g)

                    @pl.when((prev >= lo) & (prev < hi))
                    def _():
                        pltpu.make_async_copy(
                            tab_hbm.at[srcp], rowsp, sem_gp).wait()

                        def g_body(g, carry2):
                            rowids = g * L + lax.iota(jnp.int32, L)
                            sg = svp[pl.ds(g * L, L)]
                            for j in range(W):
                                colj = jnp.full((L,), j, jnp.int32)
                                v = plsc.load_gather(rowsp, [rowids, colj])
                                plsc.store_scatter(
                                    scaled, [rowids, colj], v * sg)
                            return carry2

                        lax.fori_loop(0, MCHUNK // L, g_body, 0)
                        pltpu.sync_copy(scaled, acc.at[dstp], add=True)

                    @pl.when(nxt < hi)
                    def _():
                        issue_idx(nxt, 1 - b)

                return carry

            lax.fori_loop(0, (iters + 2) // 2, chunk_pair, 0)
            plsc.subcore_barrier()

            for t in range(z_iters):
                z = sid + NS * t

                @pl.when(z < nz)
                def _():
                    pltpu.sync_copy(acc.at[pl.ds(z * zrows, zrows)],
                                    out_hbm.at[out_idx,
                                               pl.ds(z * zrows, zrows)])
            plsc.subcore_barrier()

        for core, sched in enumerate(schedules):
            @pl.when(c == core)
            def _():
                for ti, lo, hi, oi in sched:
                    do_pass(tabs[ti], lo, hi, oi)

    return k


def _tc_a(x, W1, att_src1, att_dst1):
    nb = _cdiv(N, BN)

    def body(x_ref, w_ref, as_ref, ad_ref,
             g0_ref, g1_ref, g2_ref, g3_ref, als_ref, ald_ref):
        h = jnp.dot(x_ref[...], w_ref[...],
                    preferred_element_type=jnp.float32)
        als_ref[...] = jnp.sum(h * as_ref[...], axis=1)
        ald_ref[...] = jnp.sum(h * ad_ref[...], axis=1)
        g0_ref[...] = h[:, 0:16]
        g1_ref[...] = h[:, 16:32]
        g2_ref[...] = h[:, 32:48]
        ones = jnp.ones((h.shape[0], 1), jnp.float32)
        zeros = jnp.zeros((h.shape[0], 13), jnp.float32)
        g3_ref[...] = jnp.concatenate([h[:, 48:50], ones, zeros], axis=1)

    tab = jax.ShapeDtypeStruct((N, W), jnp.float32)
    return pl.pallas_call(
        body,
        grid=(nb,),
        in_specs=[
            pl.BlockSpec((BN, 100), lambda i: (i, 0)),
            pl.BlockSpec((100, 50), lambda i: (0, 0)),
            pl.BlockSpec((1, 50), lambda i: (0, 0)),
            pl.BlockSpec((1, 50), lambda i: (0, 0)),
        ],
        out_specs=[
            pl.BlockSpec((BN, W), lambda i: (i, 0)),
            pl.BlockSpec((BN, W), lambda i: (i, 0)),
            pl.BlockSpec((BN, W), lambda i: (i, 0)),
            pl.BlockSpec((BN, W), lambda i: (i, 0)),
            pl.BlockSpec((BN,), lambda i: (i,)),
            pl.BlockSpec((BN,), lambda i: (i,)),
        ],
        out_shape=[tab, tab, tab, tab,
                   jax.ShapeDtypeStruct((N,), jnp.float32),
                   jax.ShapeDtypeStruct((N,), jnp.float32)],
    )(x, W1, att_src1.reshape(1, 50), att_dst1.reshape(1, 50))


def _tc_b(o1, b1, W2, att_src2, att_dst2):
    nb = _cdiv(N, BN)

    def body(o1_ref, b1_ref, w2_ref, as2_ref, ad2_ref,
             h2p_ref, als_ref, ald_ref):
        num = jnp.concatenate(
            [o1_ref[0], o1_ref[1], o1_ref[2], o1_ref[3][:, :2]], axis=1)
        den = o1_ref[3][:, 2:3] + 1e-16
        h1 = jnp.maximum(num / den + b1_ref[...], 0.0)
        h2 = jnp.dot(h1, w2_ref[...], preferred_element_type=jnp.float32)
        als_ref[...] = jnp.sum(h2 * as2_ref[...], axis=1)
        ald_ref[...] = jnp.sum(h2 * ad2_ref[...], axis=1)
        ones = jnp.ones((h2.shape[0], 1), jnp.float32)
        zeros = jnp.zeros((h2.shape[0], 11), jnp.float32)
        h2p_ref[...] = jnp.concatenate([h2, ones, zeros], axis=1)

    return pl.pallas_call(
        body,
        grid=(nb,),
        in_specs=[
            pl.BlockSpec((4, BN, W), lambda i: (0, i, 0)),
            pl.BlockSpec((1, 50), lambda i: (0, 0)),
            pl.BlockSpec((50, 4), lambda i: (0, 0)),
            pl.BlockSpec((1, 4), lambda i: (0, 0)),
            pl.BlockSpec((1, 4), lambda i: (0, 0)),
        ],
        out_specs=[
            pl.BlockSpec((BN, W), lambda i: (i, 0)),
            pl.BlockSpec((BN,), lambda i: (i,)),
            pl.BlockSpec((BN,), lambda i: (i,)),
        ],
        out_shape=[
            jax.ShapeDtypeStruct((N, W), jnp.float32),
            jax.ShapeDtypeStruct((N,), jnp.float32),
            jax.ShapeDtypeStruct((N,), jnp.float32),
        ],
    )(o1, b1.reshape(1, 50), W2, att_src2.reshape(1, 4),
      att_dst2.reshape(1, 4))


def _tc_c(o2, b2):
    nb = _cdiv(N, BN)

    def body(o2_ref, b2_ref, h_ref):
        s = o2_ref[0] + o2_ref[1]
        den = s[:, 4:5] + 1e-16
        h_ref[...] = jnp.maximum(s[:, :4] / den + b2_ref[...], 0.0)

    return pl.pallas_call(
        body,
        grid=(nb,),
        in_specs=[
            pl.BlockSpec((2, BN, W), lambda i: (0, i, 0)),
            pl.BlockSpec((1, 4), lambda i: (0, 0)),
        ],
        out_specs=pl.BlockSpec((BN, 4), lambda i: (i, 0)),
        out_shape=jax.ShapeDtypeStruct((N, 4), jnp.float32),
    )(o2, b2.reshape(1, 4))


_logit_kernel = _make_logit_kernel()
_msg_kernel_l1 = _make_msg_kernel(
    4,
    [[(0, 0, NCH, 0), (1, 0, NCH, 1)],
     [(2, 0, NCH, 2), (3, 0, NCH, 3)]],
    4)
_msg_kernel_l2 = _make_msg_kernel(
    1,
    [[(0, 0, NCH // 2, 0)],
     [(0, NCH // 2, NCH, 1)]],
    2)


def kernel(x, edge_index, W1, att_src1, att_dst1, b1,
           W2, att_src2, att_dst2, b2):
    src = edge_index[0]
    dst = edge_index[1]

    g0, g1, g2, g3, as1, ad1 = _tc_a(x, W1, att_src1, att_dst1)
    s1 = _logit_kernel(as1, ad1, src, dst)
    o1 = _msg_kernel_l1(g0, g1, g2, g3, src, dst, s1)

    h2p, as2, ad2 = _tc_b(o1, b1, W2, att_src2, att_dst2)
    s2 = _logit_kernel(as2, ad2, src, dst)
    o2 = _msg_kernel_l2(h2p, src, dst, s2)

    h = _tc_c(o2, b2)
    return (h, edge_index)


# speedup vs baseline: 1.6248x; 1.6248x over previous
import functools

import jax
import jax.numpy as jnp
from jax import lax
from jax.experimental import pallas as pl
from jax.experimental.pallas import tpu as pltpu
from jax.experimental.pallas import tpu_sc as plsc

N = 50000
E = 800000
NC = 2
NS = 16
L = 16
W = 16
LCHUNK = 1600
MCHUNK = 800
NCH = E // MCHUNK
BN = 2048

_SC_PARAMS = pltpu.CompilerParams(
    needs_layout_passes=False, use_tc_tiling_on_sc=False)


def _cdiv(a, b):
    return (a + b - 1) // b


def _make_logit_kernel():
    mesh = plsc.VectorSubcoreMesh(core_axis_name="c", subcore_axis_name="s")
    nw = NC * NS
    lnch = E // LCHUNK
    iters = _cdiv(lnch, nw)

    @functools.partial(
        pl.kernel,
        out_type=jax.ShapeDtypeStruct((E,), jnp.float32),
        mesh=mesh,
        compiler_params=_SC_PARAMS,
        scratch_types=[
            pltpu.VMEM((N,), jnp.float32),
            pltpu.VMEM((N,), jnp.float32),
            pltpu.VMEM((LCHUNK,), jnp.int32),
            pltpu.VMEM((LCHUNK,), jnp.int32),
            pltpu.VMEM((LCHUNK,), jnp.float32),
        ],
    )
    def k(asrc_hbm, adst_hbm, src_hbm, dst_hbm, s_hbm, asrc_v, adst_v,
          srcv, dstv, sv):
        c = lax.axis_index("c")
        sid = lax.axis_index("s")
        wid = sid * NC + c
        pltpu.sync_copy(asrc_hbm, asrc_v)
        pltpu.sync_copy(adst_hbm, adst_v)

        def chunk_body(kk, carry):
            cid = wid + nw * kk

            @pl.when(cid < lnch)
            def _():
                off = cid * LCHUNK
                pltpu.sync_copy(src_hbm.at[pl.ds(off, LCHUNK)], srcv)
                pltpu.sync_copy(dst_hbm.at[pl.ds(off, LCHUNK)], dstv)

                def g_body(g, carry2):
                    si = srcv[pl.ds(g * L, L)]
                    di = dstv[pl.ds(g * L, L)]
                    a = plsc.load_gather(asrc_v, [si])
                    b = plsc.load_gather(adst_v, [di])
                    e = a + b
                    e = jnp.where(e >= 0.0, e, 0.2 * e)
                    e = jnp.minimum(e, 60.0)
                    sv[pl.ds(g * L, L)] = jnp.exp(e)
                    return carry2

                lax.fori_loop(0, LCHUNK // L, g_body, 0)
                pltpu.sync_copy(sv, s_hbm.at[pl.ds(off, LCHUNK)])

            return carry

        lax.fori_loop(0, iters, chunk_body, 0)

    return k


def _make_msg_kernel(width, n_tables, schedules, n_out):
    mesh = plsc.VectorSubcoreMesh(core_axis_name="c", subcore_axis_name="s")
    zrows = 400
    nz = N // zrows
    z_iters = _cdiv(nz, NS)
    nblk = MCHUNK * width // 32
    shift = {32: 5, 16: 4}[width]

    scratch = [pltpu.VMEM_SHARED((N, width), jnp.bfloat16)]
    for _ in range(2):
        scratch += [
            pltpu.VMEM((MCHUNK,), jnp.int32),
            pltpu.VMEM((MCHUNK,), jnp.int32),
            pltpu.VMEM((MCHUNK,), jnp.float32),
            pltpu.VMEM((MCHUNK, width), jnp.bfloat16),
            pltpu.VMEM((MCHUNK, width), jnp.bfloat16),
            pltpu.VMEM((MCHUNK,), jnp.int32),
            pltpu.SemaphoreType.DMA,
            pltpu.SemaphoreType.DMA,
            pltpu.SemaphoreType.DMA,
        ]

    @functools.partial(
        pl.kernel,
        out_type=jax.ShapeDtypeStruct((n_out, N, width), jnp.bfloat16),
        mesh=mesh,
        compiler_params=_SC_PARAMS,
        scratch_types=scratch,
    )
    def k(*refs):
        tabs = refs[:n_tables]
        src_hbm, dst_hbm, s_hbm, out_hbm, acc = refs[n_tables:n_tables + 5]
        slots = [refs[n_tables + 5 + 9 * b:n_tables + 5 + 9 * (b + 1)]
                 for b in range(2)]
        c = lax.axis_index("c")
        sid = lax.axis_index("s")

        def issue_idx(cid, b):
            srcv, dstv, sv = slots[b][:3]
            sem_i = slots[b][6]
            off = cid * MCHUNK
            pltpu.async_copy(src_hbm.at[pl.ds(off, MCHUNK)], srcv, sem_i)
            pltpu.async_copy(dst_hbm.at[pl.ds(off, MCHUNK)], dstv, sem_i)
            pltpu.async_copy(s_hbm.at[pl.ds(off, MCHUNK)], sv, sem_i)

        def wait_idx(cid, b):
            srcv, dstv, sv = slots[b][:3]
            sem_i = slots[b][6]
            off = cid * MCHUNK
            pltpu.make_async_copy(
                src_hbm.at[pl.ds(off, MCHUNK)], srcv, sem_i).wait()
            pltpu.make_async_copy(
                dst_hbm.at[pl.ds(off, MCHUNK)], dstv, sem_i).wait()
            pltpu.make_async_copy(
                s_hbm.at[pl.ds(off, MCHUNK)], sv, sem_i).wait()

        def do_pass(tab_hbm, lo, hi, out_idx):
            iters = _cdiv(hi - lo, NS)

            z0 = slots[0][4]

            def zb32(i, carry):
                if width == 32:
                    z0[i, :] = jnp.zeros((32,), jnp.bfloat16)
                else:
                    z0[pl.ds(2 * i, 2), :] = jnp.zeros(
                        (2, 16), jnp.bfloat16)
                return carry

            lax.fori_loop(0, zrows * width // 32, zb32, 0)
            for t in range(z_iters):
                z = sid + NS * t

                @pl.when(z < nz)
                def _():
                    pltpu.sync_copy(z0.at[pl.ds(0, zrows)],
                                    acc.at[pl.ds(z * zrows, zrows)])
            plsc.subcore_barrier()

            cid0 = lo + sid

            @pl.when(cid0 < hi)
            def _():
                issue_idx(cid0, 0)

            def chunk_pair(kk2, carry):
                for b in (0, 1):
                    kk2b = 2 * kk2 + b
                    cur = lo + sid + NS * kk2b
                    prv = cur - NS
                    old = cur - 3 * NS
                    srcv, dstv, sv, rows, scaled, dss, sem_i, sem_g, \
                        sem_s = slots[b]
                    srcp, dstp, svp, rowsp, scaledp, dssp, _, sem_gp, \
                        sem_sp = slots[1 - b]

                    @pl.when(cur < hi)
                    def _():
                        wait_idx(cur, b)
                        pltpu.async_copy(tab_hbm.at[srcv], rows, sem_g)

                    @pl.when((old >= lo) & (old < hi))
                    def _():
                        pltpu.make_async_copy(
                            scaledp, acc.at[dssp], sem_sp).wait()

                    @pl.when((prv >= lo) & (prv < hi))
                    def _():
                        pltpu.make_async_copy(
                            tab_hbm.at[srcp], rowsp, sem_gp).wait()

                        def g_body(i, carry2):
                            if width == 32:
                                v = rowsp[i, :]
                            else:
                                v = rowsp[pl.ds(2 * i, 2), :].reshape(32)
                            eidx = (32 * i + 2 * lax.iota(jnp.int32, L)
                                    ) >> shift
                            sg = plsc.load_gather(svp, [eidx])
                            a, bb = plsc.unpack(
                                v, format=plsc.PackFormat.INTERLEAVED,
                                preferred_element_type=jnp.float32)
                            out = plsc.pack(
                                a * sg, bb * sg,
                                format=plsc.PackFormat.INTERLEAVED)
                            if width == 32:
                                scaledp[i, :] = out
                            else:
                                scaledp[pl.ds(2 * i, 2), :] = out.reshape(
                                    (2, 16))
                            return carry2

                        lax.fori_loop(0, nblk, g_body, 0)

                        def cp_body(i, carry2):
                            dssp[pl.ds(i * L, L)] = dstp[pl.ds(i * L, L)]
                            return carry2

                        lax.fori_loop(0, MCHUNK // L, cp_body, 0)
                        pltpu.async_copy(
                            scaledp, acc.at[dssp], sem_sp, add=True)

                    @pl.when(cur + NS < hi)
                    def _():
                        issue_idx(cur + NS, 1 - b)

                return carry

            lax.fori_loop(0, (iters + 4) // 2, chunk_pair, 0)
            plsc.subcore_barrier()

            for t in range(z_iters):
                z = sid + NS * t

                @pl.when(z < nz)
                def _():
                    pltpu.sync_copy(acc.at[pl.ds(z * zrows, zrows)],
                                    out_hbm.at[out_idx,
                                               pl.ds(z * zrows, zrows)])
            plsc.subcore_barrier()

        for core, sched in enumerate(schedules):
            @pl.when(c == core)
            def _():
                for ti, lo, hi, oi in sched:
                    do_pass(tabs[ti], lo, hi, oi)

    return k


def _tc_a(x, W1, att_src1, att_dst1):
    nb = _cdiv(N, BN)

    def body(x_ref, w_ref, as_ref, ad_ref, g0_ref, g1_ref, als_ref, ald_ref):
        h = jnp.dot(x_ref[...], w_ref[...],
                    preferred_element_type=jnp.float32)
        als_ref[...] = jnp.sum(h * as_ref[...], axis=1)
        ald_ref[...] = jnp.sum(h * ad_ref[...], axis=1)
        g0_ref[...] = h[:, 0:32].astype(jnp.bfloat16)
        ones = jnp.ones((h.shape[0], 1), jnp.float32)
        zeros = jnp.zeros((h.shape[0], 13), jnp.float32)
        g1_ref[...] = jnp.concatenate(
            [h[:, 32:50], ones, zeros], axis=1).astype(jnp.bfloat16)

    tab = jax.ShapeDtypeStruct((N, 32), jnp.bfloat16)
    return pl.pallas_call(
        body,
        grid=(nb,),
        in_specs=[
            pl.BlockSpec((BN, 100), lambda i: (i, 0)),
            pl.BlockSpec((100, 50), lambda i: (0, 0)),
            pl.BlockSpec((1, 50), lambda i: (0, 0)),
            pl.BlockSpec((1, 50), lambda i: (0, 0)),
        ],
        out_specs=[
            pl.BlockSpec((BN, 32), lambda i: (i, 0)),
            pl.BlockSpec((BN, 32), lambda i: (i, 0)),
            pl.BlockSpec((BN,), lambda i: (i,)),
            pl.BlockSpec((BN,), lambda i: (i,)),
        ],
        out_shape=[tab, tab,
                   jax.ShapeDtypeStruct((N,), jnp.float32),
                   jax.ShapeDtypeStruct((N,), jnp.float32)],
    )(x, W1, att_src1.reshape(1, 50), att_dst1.reshape(1, 50))


def _tc_b(o1, b1, W2, att_src2, att_dst2):
    nb = _cdiv(N, BN)

    def body(o1_ref, b1_ref, w2_ref, as2_ref, ad2_ref,
             h2p_ref, als_ref, ald_ref):
        num = jnp.concatenate(
            [o1_ref[0], o1_ref[1][:, :18]], axis=1).astype(jnp.float32)
        den = o1_ref[1][:, 18:19].astype(jnp.float32) + 1e-16
        h1 = jnp.maximum(num / den + b1_ref[...], 0.0)
        h2 = jnp.dot(h1, w2_ref[...], preferred_element_type=jnp.float32)
        als_ref[...] = jnp.sum(h2 * as2_ref[...], axis=1)
        ald_ref[...] = jnp.sum(h2 * ad2_ref[...], axis=1)
        ones = jnp.ones((h2.shape[0], 1), jnp.float32)
        zeros = jnp.zeros((h2.shape[0], 27), jnp.float32)
        h2p_ref[...] = jnp.concatenate(
            [h2, ones, zeros], axis=1).astype(jnp.bfloat16)

    return pl.pallas_call(
        body,
        grid=(nb,),
        in_specs=[
            pl.BlockSpec((2, BN, 32), lambda i: (0, i, 0)),
            pl.BlockSpec((1, 50), lambda i: (0, 0)),
            pl.BlockSpec((50, 4), lambda i: (0, 0)),
            pl.BlockSpec((1, 4), lambda i: (0, 0)),
            pl.BlockSpec((1, 4), lambda i: (0, 0)),
        ],
        out_specs=[
            pl.BlockSpec((BN, 32), lambda i: (i, 0)),
            pl.BlockSpec((BN,), lambda i: (i,)),
            pl.BlockSpec((BN,), lambda i: (i,)),
        ],
        out_shape=[
            jax.ShapeDtypeStruct((N, 32), jnp.bfloat16),
            jax.ShapeDtypeStruct((N,), jnp.float32),
            jax.ShapeDtypeStruct((N,), jnp.float32),
        ],
    )(o1, b1.reshape(1, 50), W2, att_src2.reshape(1, 4),
      att_dst2.reshape(1, 4))


def _tc_c(o2, b2):
    nb = _cdiv(N, BN)

    def body(o2_ref, b2_ref, h_ref):
        s = o2_ref[0].astype(jnp.float32) + o2_ref[1].astype(jnp.float32)
        den = s[:, 4:5] + 1e-16
        h_ref[...] = jnp.maximum(s[:, :4] / den + b2_ref[...], 0.0)

    return pl.pallas_call(
        body,
        grid=(nb,),
        in_specs=[
            pl.BlockSpec((2, BN, 32), lambda i: (0, i, 0)),
            pl.BlockSpec((1, 4), lambda i: (0, 0)),
        ],
        out_specs=pl.BlockSpec((BN, 4), lambda i: (i, 0)),
        out_shape=jax.ShapeDtypeStruct((N, 4), jnp.float32),
    )(o2, b2.reshape(1, 4))


_logit_kernel = _make_logit_kernel()
_msg_kernel_l1 = _make_msg_kernel(
    32, 2,
    [[(0, 0, NCH, 0)],
     [(1, 0, NCH, 1)]],
    2)
_msg_kernel_l2 = _make_msg_kernel(
    32, 1,
    [[(0, 0, NCH // 2, 0)],
     [(0, NCH // 2, NCH, 1)]],
    2)


def kernel(x, edge_index, W1, att_src1, att_dst1, b1,
           W2, att_src2, att_dst2, b2):
    src = edge_index[0]
    dst = edge_index[1]

    g0, g1, as1, ad1 = _tc_a(x, W1, att_src1, att_dst1)
    s1 = _logit_kernel(as1, ad1, src, dst)
    o1 = _msg_kernel_l1(g0, g1, src, dst, s1)

    h2p, as2, ad2 = _tc_b(o1, b1, W2, att_src2, att_dst2)
    s2 = _logit_kernel(as2, ad2, src, dst)
    o2 = _msg_kernel_l2(h2p, src, dst, s2)

    h = _tc_c(o2, b2)
    return (h, edge_index)


# speedup vs baseline: 1.6690x; 1.0272x over previous
import functools

import jax
import jax.numpy as jnp
from jax import lax
from jax.experimental import pallas as pl
from jax.experimental.pallas import tpu as pltpu
from jax.experimental.pallas import tpu_sc as plsc

N = 50000
E = 800000
NC = 2
NS = 16
L = 16
W = 16
LCHUNK = 6400
MCHUNK = 800
NCH = E // MCHUNK
BN = 2048

_SC_PARAMS = pltpu.CompilerParams(
    needs_layout_passes=False, use_tc_tiling_on_sc=False)


def _cdiv(a, b):
    return (a + b - 1) // b


def _make_logit_kernel():
    mesh = plsc.VectorSubcoreMesh(core_axis_name="c", subcore_axis_name="s")
    nw = NC * NS
    lnch = E // LCHUNK
    iters = _cdiv(lnch, nw)

    @functools.partial(
        pl.kernel,
        out_type=jax.ShapeDtypeStruct((E,), jnp.float32),
        mesh=mesh,
        compiler_params=_SC_PARAMS,
        scratch_types=[
            pltpu.VMEM((N,), jnp.float32),
            pltpu.VMEM((N,), jnp.float32),
            pltpu.VMEM((LCHUNK,), jnp.int32),
            pltpu.VMEM((LCHUNK,), jnp.int32),
            pltpu.VMEM((LCHUNK,), jnp.float32),
        ],
    )
    def k(asrc_hbm, adst_hbm, src_hbm, dst_hbm, s_hbm, asrc_v, adst_v,
          srcv, dstv, sv):
        c = lax.axis_index("c")
        sid = lax.axis_index("s")
        wid = sid * NC + c
        pltpu.sync_copy(asrc_hbm, asrc_v)
        pltpu.sync_copy(adst_hbm, adst_v)

        def chunk_body(kk, carry):
            cid = wid + nw * kk

            @pl.when(cid < lnch)
            def _():
                off = cid * LCHUNK
                pltpu.sync_copy(src_hbm.at[pl.ds(off, LCHUNK)], srcv)
                pltpu.sync_copy(dst_hbm.at[pl.ds(off, LCHUNK)], dstv)

                def g_body(g, carry2):
                    si = srcv[pl.ds(g * L, L)]
                    di = dstv[pl.ds(g * L, L)]
                    a = plsc.load_gather(asrc_v, [si])
                    b = plsc.load_gather(adst_v, [di])
                    e = a + b
                    e = jnp.where(e >= 0.0, e, 0.2 * e)
                    e = jnp.minimum(e, 60.0)
                    sv[pl.ds(g * L, L)] = jnp.exp(e)
                    return carry2

                lax.fori_loop(0, LCHUNK // L, g_body, 0)
                pltpu.sync_copy(sv, s_hbm.at[pl.ds(off, LCHUNK)])

            return carry

        lax.fori_loop(0, iters, chunk_body, 0)

    return k


def _make_msg_kernel(width, n_tables, schedules, n_out):
    mesh = plsc.VectorSubcoreMesh(core_axis_name="c", subcore_axis_name="s")
    zrows = 400
    nz = N // zrows
    z_iters = _cdiv(nz, NS)
    nblk = MCHUNK * width // 32

    scratch = [pltpu.VMEM_SHARED((N, width), jnp.bfloat16)]
    for _ in range(2):
        scratch += [
            pltpu.VMEM((MCHUNK,), jnp.int32),
            pltpu.VMEM((MCHUNK,), jnp.int32),
            pltpu.VMEM((MCHUNK,), jnp.float32),
            pltpu.VMEM((MCHUNK, width), jnp.bfloat16),
            pltpu.VMEM((MCHUNK, width), jnp.bfloat16),
            pltpu.VMEM((MCHUNK,), jnp.int32),
            pltpu.SemaphoreType.DMA,
            pltpu.SemaphoreType.DMA,
            pltpu.SemaphoreType.DMA,
        ]

    @functools.partial(
        pl.kernel,
        out_type=jax.ShapeDtypeStruct((n_out, N, width), jnp.bfloat16),
        mesh=mesh,
        compiler_params=_SC_PARAMS,
        scratch_types=scratch,
    )
    def k(*refs):
        tabs = refs[:n_tables]
        src_hbm, dst_hbm, s_hbm, out_hbm, acc = refs[n_tables:n_tables + 5]
        slots = [refs[n_tables + 5 + 9 * b:n_tables + 5 + 9 * (b + 1)]
                 for b in range(2)]
        c = lax.axis_index("c")
        sid = lax.axis_index("s")

        def issue_idx(cid, b):
            srcv, dstv, sv = slots[b][:3]
            sem_i = slots[b][6]
            off = cid * MCHUNK
            pltpu.async_copy(src_hbm.at[pl.ds(off, MCHUNK)], srcv, sem_i)
            pltpu.async_copy(dst_hbm.at[pl.ds(off, MCHUNK)], dstv, sem_i)
            pltpu.async_copy(s_hbm.at[pl.ds(off, MCHUNK)], sv, sem_i)

        def wait_idx(cid, b):
            srcv, dstv, sv = slots[b][:3]
            sem_i = slots[b][6]
            off = cid * MCHUNK
            pltpu.make_async_copy(
                src_hbm.at[pl.ds(off, MCHUNK)], srcv, sem_i).wait()
            pltpu.make_async_copy(
                dst_hbm.at[pl.ds(off, MCHUNK)], dstv, sem_i).wait()
            pltpu.make_async_copy(
                s_hbm.at[pl.ds(off, MCHUNK)], sv, sem_i).wait()

        def do_pass(tab_hbm, lo, hi, out_idx):
            iters = _cdiv(hi - lo, NS)

            z0 = slots[0][4]

            def zb32(i, carry):
                if width == 32:
                    z0[i, :] = jnp.zeros((32,), jnp.bfloat16)
                else:
                    z0[pl.ds(2 * i, 2), :] = jnp.zeros(
                        (2, 16), jnp.bfloat16)
                return carry

            lax.fori_loop(0, zrows * width // 32, zb32, 0)
            for t in range(z_iters):
                z = sid + NS * t

                @pl.when(z < nz)
                def _():
                    pltpu.sync_copy(z0.at[pl.ds(0, zrows)],
                                    acc.at[pl.ds(z * zrows, zrows)])
            plsc.subcore_barrier()

            cid0 = lo + sid

            @pl.when(cid0 < hi)
            def _():
                issue_idx(cid0, 0)

            def chunk_pair(kk2, carry):
                for b in (0, 1):
                    kk2b = 2 * kk2 + b
                    cur = lo + sid + NS * kk2b
                    prv = cur - NS
                    old = cur - 3 * NS
                    srcv, dstv, sv, rows, scaled, dss, sem_i, sem_g, \
                        sem_s = slots[b]
                    srcp, dstp, svp, rowsp, scaledp, dssp, _, sem_gp, \
                        sem_sp = slots[1 - b]

                    @pl.when(cur < hi)
                    def _():
                        wait_idx(cur, b)
                        pltpu.async_copy(tab_hbm.at[srcv], rows, sem_g)

                    @pl.when((old >= lo) & (old < hi))
                    def _():
                        pltpu.make_async_copy(
                            scaledp, acc.at[dssp], sem_sp).wait()

                    @pl.when((prv >= lo) & (prv < hi))
                    def _():
                        pltpu.make_async_copy(
                            tab_hbm.at[srcp], rowsp, sem_gp).wait()

                        def g_body(i, carry2):
                            v = rowsp[i, :]
                            eidx = jnp.full((L,), i, jnp.int32)
                            sg = plsc.load_gather(svp, [eidx])
                            a, bb = plsc.unpack(
                                v, format=plsc.PackFormat.INTERLEAVED,
                                preferred_element_type=jnp.float32)
                            scaledp[i, :] = plsc.pack(
                                a * sg, bb * sg,
                                format=plsc.PackFormat.INTERLEAVED)
                            return carry2

                        lax.fori_loop(0, nblk, g_body, 0)

                        def cp_body(i, carry2):
                            dssp[pl.ds(i * L, L)] = dstp[pl.ds(i * L, L)]
                            return carry2

                        lax.fori_loop(0, MCHUNK // L, cp_body, 0)
                        pltpu.async_copy(
                            scaledp, acc.at[dssp], sem_sp, add=True)

                    @pl.when(cur + NS < hi)
                    def _():
                        issue_idx(cur + NS, 1 - b)

                return carry

            lax.fori_loop(0, (iters + 4) // 2, chunk_pair, 0)
            plsc.subcore_barrier()

            for t in range(z_iters):
                z = sid + NS * t

                @pl.when(z < nz)
                def _():
                    pltpu.sync_copy(acc.at[pl.ds(z * zrows, zrows)],
                                    out_hbm.at[out_idx,
                                               pl.ds(z * zrows, zrows)])
            plsc.subcore_barrier()

        for core, sched in enumerate(schedules):
            @pl.when(c == core)
            def _():
                for ti, lo, hi, oi in sched:
                    do_pass(tabs[ti], lo, hi, oi)

    return k


def _tc_a(x, W1, att_src1, att_dst1):
    nb = _cdiv(N, BN)

    def body(x_ref, w_ref, as_ref, ad_ref, g0_ref, g1_ref, als_ref, ald_ref):
        h = jnp.dot(x_ref[...], w_ref[...],
                    preferred_element_type=jnp.float32)
        als_ref[...] = jnp.sum(h * as_ref[...], axis=1)
        ald_ref[...] = jnp.sum(h * ad_ref[...], axis=1)
        g0_ref[...] = h[:, 0:32].astype(jnp.bfloat16)
        ones = jnp.ones((h.shape[0], 1), jnp.float32)
        zeros = jnp.zeros((h.shape[0], 13), jnp.float32)
        g1_ref[...] = jnp.concatenate(
            [h[:, 32:50], ones, zeros], axis=1).astype(jnp.bfloat16)

    tab = jax.ShapeDtypeStruct((N, 32), jnp.bfloat16)
    return pl.pallas_call(
        body,
        grid=(nb,),
        in_specs=[
            pl.BlockSpec((BN, 100), lambda i: (i, 0)),
            pl.BlockSpec((100, 50), lambda i: (0, 0)),
            pl.BlockSpec((1, 50), lambda i: (0, 0)),
            pl.BlockSpec((1, 50), lambda i: (0, 0)),
        ],
        out_specs=[
            pl.BlockSpec((BN, 32), lambda i: (i, 0)),
            pl.BlockSpec((BN, 32), lambda i: (i, 0)),
            pl.BlockSpec((BN,), lambda i: (i,)),
            pl.BlockSpec((BN,), lambda i: (i,)),
        ],
        out_shape=[tab, tab,
                   jax.ShapeDtypeStruct((N,), jnp.float32),
                   jax.ShapeDtypeStruct((N,), jnp.float32)],
    )(x, W1, att_src1.reshape(1, 50), att_dst1.reshape(1, 50))


def _tc_b(o1, b1, W2, att_src2, att_dst2):
    nb = _cdiv(N, BN)

    def body(o1_ref, b1_ref, w2_ref, as2_ref, ad2_ref,
             h2p_ref, als_ref, ald_ref):
        num = jnp.concatenate(
            [o1_ref[0], o1_ref[1][:, :18]], axis=1).astype(jnp.float32)
        den = o1_ref[1][:, 18:19].astype(jnp.float32) + 1e-16
        h1 = jnp.maximum(num / den + b1_ref[...], 0.0)
        h2 = jnp.dot(h1, w2_ref[...], preferred_element_type=jnp.float32)
        als_ref[...] = jnp.sum(h2 * as2_ref[...], axis=1)
        ald_ref[...] = jnp.sum(h2 * ad2_ref[...], axis=1)
        ones = jnp.ones((h2.shape[0], 1), jnp.float32)
        zeros = jnp.zeros((h2.shape[0], 27), jnp.float32)
        h2p_ref[...] = jnp.concatenate(
            [h2, ones, zeros], axis=1).astype(jnp.bfloat16)

    return pl.pallas_call(
        body,
        grid=(nb,),
        in_specs=[
            pl.BlockSpec((2, BN, 32), lambda i: (0, i, 0)),
            pl.BlockSpec((1, 50), lambda i: (0, 0)),
            pl.BlockSpec((50, 4), lambda i: (0, 0)),
            pl.BlockSpec((1, 4), lambda i: (0, 0)),
            pl.BlockSpec((1, 4), lambda i: (0, 0)),
        ],
        out_specs=[
            pl.BlockSpec((BN, 32), lambda i: (i, 0)),
            pl.BlockSpec((BN,), lambda i: (i,)),
            pl.BlockSpec((BN,), lambda i: (i,)),
        ],
        out_shape=[
            jax.ShapeDtypeStruct((N, 32), jnp.bfloat16),
            jax.ShapeDtypeStruct((N,), jnp.float32),
            jax.ShapeDtypeStruct((N,), jnp.float32),
        ],
    )(o1, b1.reshape(1, 50), W2, att_src2.reshape(1, 4),
      att_dst2.reshape(1, 4))


def _tc_c(o2, b2):
    nb = _cdiv(N, BN)

    def body(o2_ref, b2_ref, h_ref):
        s = o2_ref[0].astype(jnp.float32) + o2_ref[1].astype(jnp.float32)
        den = s[:, 4:5] + 1e-16
        h_ref[...] = jnp.maximum(s[:, :4] / den + b2_ref[...], 0.0)

    return pl.pallas_call(
        body,
        grid=(nb,),
        in_specs=[
            pl.BlockSpec((2, BN, 32), lambda i: (0, i, 0)),
            pl.BlockSpec((1, 4), lambda i: (0, 0)),
        ],
        out_specs=pl.BlockSpec((BN, 4), lambda i: (i, 0)),
        out_shape=jax.ShapeDtypeStruct((N, 4), jnp.float32),
    )(o2, b2.reshape(1, 4))


_logit_kernel = _make_logit_kernel()
_msg_kernel_l1 = _make_msg_kernel(
    32, 2,
    [[(0, 0, NCH, 0)],
     [(1, 0, NCH, 1)]],
    2)
_msg_kernel_l2 = _make_msg_kernel(
    32, 1,
    [[(0, 0, NCH // 2, 0)],
     [(0, NCH // 2, NCH, 1)]],
    2)


def kernel(x, edge_index, W1, att_src1, att_dst1, b1,
           W2, att_src2, att_dst2, b2):
    src = edge_index[0]
    dst = edge_index[1]

    g0, g1, as1, ad1 = _tc_a(x, W1, att_src1, att_dst1)
    s1 = _logit_kernel(as1, ad1, src, dst)
    o1 = _msg_kernel_l1(g0, g1, src, dst, s1)

    h2p, as2, ad2 = _tc_b(o1, b1, W2, att_src2, att_dst2)
    s2 = _logit_kernel(as2, ad2, src, dst)
    o2 = _msg_kernel_l2(h2p, src, dst, s2)

    h = _tc_c(o2, b2)
    return (h, edge_index)
